# Initial kernel scaffold; baseline (speedup 1.0000x reference)
import jax
import jax.numpy as jnp
from jax.experimental import pallas as pl


def kernel(x):
    raise NotImplementedError("write your pallas kernel here")


# speedup vs baseline: 42.8401x; 42.8401x over previous
import functools

import jax
import jax.numpy as jnp
from jax import lax
from jax.experimental import pallas as pl
from jax.experimental.pallas import tpu as pltpu
from jax.experimental.pallas import tpu_sc as plsc

BINS = 512
L = 16
NC = 2
NS = 16
NW = NC * NS

TOTAL = 4 * 8192 * 2048
TILE_N = TOTAL // NW
CHUNK = 32768
NCHUNK = TILE_N // CHUNK
VECS = CHUNK // L


def _minmax_body(x_ref, mn_ref, mx_ref):
    i = pl.program_id(0)
    blk = x_ref[...]
    bmn = jnp.min(blk)
    bmx = jnp.max(blk)

    @pl.when(i == 0)
    def _():
        mn_ref[0, 0] = bmn
        mx_ref[0, 0] = bmx

    @pl.when(i > 0)
    def _():
        mn_ref[0, 0] = jnp.minimum(mn_ref[0, 0], bmn)
        mx_ref[0, 0] = jnp.maximum(mx_ref[0, 0], bmx)


def _minmax(x2d):
    rows = x2d.shape[0]
    blk_rows = 512
    grid = rows // blk_rows
    return pl.pallas_call(
        _minmax_body,
        grid=(grid,),
        in_specs=[pl.BlockSpec((blk_rows, x2d.shape[1]), lambda i: (i, 0))],
        out_specs=[
            pl.BlockSpec(memory_space=pltpu.SMEM),
            pl.BlockSpec(memory_space=pltpu.SMEM),
        ],
        out_shape=[
            jax.ShapeDtypeStruct((1, 1), jnp.float32),
            jax.ShapeDtypeStruct((1, 1), jnp.float32),
        ],
    )(x2d)


def _hist_sc_body(x_hbm, minv_hbm, invw_hbm, out_hbm,
                  buf0, buf1, prm0, prm1, hist, part, sem0, sem1):
    cid = lax.axis_index("c")
    sid = lax.axis_index("s")
    wid = sid * NC + cid
    base = wid * TILE_N

    pltpu.sync_copy(minv_hbm, prm0)
    pltpu.sync_copy(invw_hbm, prm1)
    minv = prm0[...]
    invw = prm1[...]
    nbias = jnp.zeros((L,), jnp.float32) - minv * invw
    lane_base = lax.iota(jnp.int32, L) * BINS
    ones = jnp.full((L,), 1.0, jnp.float32)
    c511 = jnp.full((L,), BINS - 1, jnp.int32)

    def _zero(j, c):
        hist[pl.ds(j * L, L)] = jnp.zeros((L,), jnp.float32)
        return c

    lax.fori_loop(0, (L * BINS) // L, _zero, 0)

    pltpu.make_async_copy(x_hbm.at[pl.ds(base, CHUNK)], buf0, sem0).start()
    pltpu.make_async_copy(
        x_hbm.at[pl.ds(base + CHUNK, CHUNK)], buf1, sem1).start()

    def _process(buf):
        def _vec(i, c):
            v = buf[pl.ds(i * L, L)]
            q = v * invw + nbias
            iv = jnp.minimum(q.astype(jnp.int32), c511)
            addr = iv + lane_base
            plsc.addupdate_scatter(hist, [addr], ones)
            return c

        lax.fori_loop(0, VECS, _vec, 0)

    def _chunk_pair(g, c):
        off_a = base + (2 * g) * CHUNK
        pltpu.make_async_copy(x_hbm.at[pl.ds(off_a, CHUNK)], buf0, sem0).wait()
        _process(buf0)

        @pl.when(g < NCHUNK // 2 - 1)
        def _():
            pltpu.make_async_copy(
                x_hbm.at[pl.ds(off_a + 2 * CHUNK, CHUNK)], buf0, sem0).start()

        off_b = base + (2 * g + 1) * CHUNK
        pltpu.make_async_copy(x_hbm.at[pl.ds(off_b, CHUNK)], buf1, sem1).wait()
        _process(buf1)

        @pl.when(g < NCHUNK // 2 - 1)
        def _():
            pltpu.make_async_copy(
                x_hbm.at[pl.ds(off_b + 2 * CHUNK, CHUNK)], buf1, sem1).start()

        return c

    lax.fori_loop(0, NCHUNK // 2, _chunk_pair, 0)

    def _red(j, c):
        def _lane(l, acc):
            return acc + hist[pl.ds(l * BINS + j * L, L)]

        acc = lax.fori_loop(1, L, _lane, hist[pl.ds(j * L, L)])
        part[pl.ds(j * L, L)] = acc
        return c

    lax.fori_loop(0, BINS // L, _red, 0)

    pltpu.sync_copy(part, out_hbm.at[wid])


def _hist_sc(x_flat, minv, invw):
    mesh = plsc.VectorSubcoreMesh(
        core_axis_name="c", subcore_axis_name="s",
        num_cores=NC, num_subcores=NS)
    call = functools.partial(
        pl.kernel,
        out_type=jax.ShapeDtypeStruct((NW, BINS), jnp.float32),
        mesh=mesh,
        compiler_params=pltpu.CompilerParams(needs_layout_passes=False),
        scratch_types=[
            pltpu.VMEM((CHUNK,), jnp.float32),
            pltpu.VMEM((CHUNK,), jnp.float32),
            pltpu.VMEM((L,), jnp.float32),
            pltpu.VMEM((L,), jnp.float32),
            pltpu.VMEM((L * BINS,), jnp.float32),
            pltpu.VMEM((BINS,), jnp.float32),
            pltpu.SemaphoreType.DMA,
            pltpu.SemaphoreType.DMA,
        ],
    )(_hist_sc_body)
    return call(x_flat, minv, invw)


def kernel(x):
    x2d = x.reshape(4 * 8192, 2048)
    mn, mx = _minmax(x2d)
    min_val = mn[0, 0]
    max_val = mx[0, 0]
    width = (max_val - min_val) / jnp.float32(BINS)
    safe_width = jnp.where(width > 0, width, jnp.float32(1.0))
    invw = jnp.float32(1.0) / safe_width
    minv = jnp.full((L,), min_val, jnp.float32)
    invwv = jnp.full((L,), invw, jnp.float32)
    parts = _hist_sc(x.reshape(TOTAL), minv, invwv)
    hist = parts.sum(axis=0)
    return (x, hist, min_val, max_val)


# speedup vs baseline: 164.2497x; 3.8340x over previous
import functools

import jax
import jax.numpy as jnp
from jax import lax
from jax.experimental import pallas as pl
from jax.experimental.pallas import tpu as pltpu
from jax.experimental.pallas import tpu_sc as plsc

BINS = 512
L = 16
NC = 2
NS = 16
NW = NC * NS

TOTAL = 4 * 8192 * 2048
TILE_N = TOTAL // NW
NBUF = 4
CHUNK = 16384
NCHUNK = TILE_N // CHUNK
VECS = CHUNK // L


def _minmax_body(x_ref, xo_ref, mn_ref, mx_ref):
    i = pl.program_id(0)
    blk = x_ref[...]
    xo_ref[...] = blk
    bmn = jnp.min(blk)
    bmx = jnp.max(blk)

    @pl.when(i == 0)
    def _():
        mn_ref[0, 0] = bmn
        mx_ref[0, 0] = bmx

    @pl.when(i > 0)
    def _():
        mn_ref[0, 0] = jnp.minimum(mn_ref[0, 0], bmn)
        mx_ref[0, 0] = jnp.maximum(mx_ref[0, 0], bmx)


def _minmax_copy(x2d):
    rows, cols = x2d.shape
    blk_rows = 512
    grid = rows // blk_rows
    return pl.pallas_call(
        _minmax_body,
        grid=(grid,),
        in_specs=[pl.BlockSpec((blk_rows, cols), lambda i: (i, 0))],
        out_specs=[
            pl.BlockSpec((blk_rows, cols), lambda i: (i, 0)),
            pl.BlockSpec(memory_space=pltpu.SMEM),
            pl.BlockSpec(memory_space=pltpu.SMEM),
        ],
        out_shape=[
            jax.ShapeDtypeStruct((rows, cols), jnp.float32),
            jax.ShapeDtypeStruct((1, 1), jnp.float32),
            jax.ShapeDtypeStruct((1, 1), jnp.float32),
        ],
    )(x2d)


def _hist_sc_body(x_hbm, minv_hbm, invw_hbm, out_hbm,
                  buf0, buf1, buf2, buf3, prm0, prm1, hist, part,
                  sem0, sem1, sem2, sem3):
    bufs = (buf0, buf1, buf2, buf3)
    sems = (sem0, sem1, sem2, sem3)
    cid = lax.axis_index("c")
    sid = lax.axis_index("s")
    wid = sid * NC + cid
    base = wid * TILE_N

    pltpu.sync_copy(minv_hbm, prm0)
    pltpu.sync_copy(invw_hbm, prm1)
    minv = prm0[...]
    invw = prm1[...]
    nbias = jnp.zeros((L,), jnp.float32) - minv * invw
    lane = lax.iota(jnp.int32, L)
    lane16 = lane * L
    ones = jnp.full((L,), 1.0, jnp.float32)
    chi = jnp.full((L,), BINS - 1 + 0.5, jnp.float32)

    @plsc.parallel_loop(0, (L * BINS) // L, 1, unroll=4)
    def _zero(j):
        hist[pl.ds(j * L, L)] = jnp.zeros((L,), jnp.float32)

    for b in range(NBUF):
        pltpu.make_async_copy(
            x_hbm.at[pl.ds(base + b * CHUNK, CHUNK)], bufs[b], sems[b]).start()

    def _process(buf):
        @plsc.parallel_loop(0, VECS, 1, unroll=8)
        def _vec(i):
            v = buf[pl.ds(i * L, L)]
            q = v * invw + nbias
            iv = jnp.minimum(q, chi).astype(jnp.int32)
            addr = (iv << 4) + lane
            plsc.addupdate_scatter(hist, [addr], ones)

    def _round(g, c):
        for b in range(NBUF):
            ch = g * NBUF + b
            off = base + ch * CHUNK
            pltpu.make_async_copy(
                x_hbm.at[pl.ds(off, CHUNK)], bufs[b], sems[b]).wait()
            _process(bufs[b])

            @pl.when(ch + NBUF < NCHUNK)
            def _():
                pltpu.make_async_copy(
                    x_hbm.at[pl.ds(off + NBUF * CHUNK, CHUNK)],
                    bufs[b], sems[b]).start()

        return c

    lax.fori_loop(0, NCHUNK // NBUF, _round, 0)

    def _red(j, c):
        def _lane(l, acc):
            return acc + plsc.load_gather(hist, [lane16 + (j * (L * L) + l)])

        acc = lax.fori_loop(1, L, _lane,
                            plsc.load_gather(hist, [lane16 + j * (L * L)]))
        part[pl.ds(j * L, L)] = acc
        return c

    lax.fori_loop(0, BINS // L, _red, 0)

    pltpu.sync_copy(part, out_hbm.at[wid])


def _hist_sc(x_flat, minv, invw):
    mesh = plsc.VectorSubcoreMesh(
        core_axis_name="c", subcore_axis_name="s",
        num_cores=NC, num_subcores=NS)
    call = functools.partial(
        pl.kernel,
        out_type=jax.ShapeDtypeStruct((NW, BINS), jnp.float32),
        mesh=mesh,
        compiler_params=pltpu.CompilerParams(needs_layout_passes=False),
        scratch_types=[
            pltpu.VMEM((CHUNK,), jnp.float32),
            pltpu.VMEM((CHUNK,), jnp.float32),
            pltpu.VMEM((CHUNK,), jnp.float32),
            pltpu.VMEM((CHUNK,), jnp.float32),
            pltpu.VMEM((L,), jnp.float32),
            pltpu.VMEM((L,), jnp.float32),
            pltpu.VMEM((L * BINS,), jnp.float32),
            pltpu.VMEM((BINS,), jnp.float32),
            pltpu.SemaphoreType.DMA,
            pltpu.SemaphoreType.DMA,
            pltpu.SemaphoreType.DMA,
            pltpu.SemaphoreType.DMA,
        ],
    )(_hist_sc_body)
    return call(x_flat, minv, invw)


def kernel(x):
    x2d = x.reshape(4 * 8192, 2048)
    x_out, mn, mx = _minmax_copy(x2d)
    min_val = mn[0, 0]
    max_val = mx[0, 0]
    width = (max_val - min_val) / jnp.float32(BINS)
    safe_width = jnp.where(width > 0, width, jnp.float32(1.0))
    invw = jnp.float32(1.0) / safe_width
    minv = jnp.full((L,), min_val, jnp.float32)
    invwv = jnp.full((L,), invw, jnp.float32)
    parts = _hist_sc(x.reshape(TOTAL), minv, invwv)
    hist = parts.sum(axis=0)
    return (x_out.reshape(x.shape), hist, min_val, max_val)


# speedup vs baseline: 228.2548x; 1.3897x over previous
import functools

import jax
import jax.numpy as jnp
from jax import lax
from jax.experimental import pallas as pl
from jax.experimental.pallas import tpu as pltpu
from jax.experimental.pallas import tpu_sc as plsc

BINS = 512
L = 16
NC = 2
NS = 16
NW = NC * NS

ROWS = 4 * 8192
COLS = 2048
ROWS_PER_TILE = ROWS // NW
NBUF = 4
CHUNK_ROWS = 8
NCHUNK = ROWS_PER_TILE // CHUNK_ROWS
VECS = COLS // L


def _minmax_body(x_ref, mn_ref, mx_ref):
    i = pl.program_id(0)
    blk = x_ref[...]
    bmn = jnp.min(blk)
    bmx = jnp.max(blk)

    @pl.when(i == 0)
    def _():
        mn_ref[0, 0] = bmn
        mx_ref[0, 0] = bmx

    @pl.when(i > 0)
    def _():
        mn_ref[0, 0] = jnp.minimum(mn_ref[0, 0], bmn)
        mx_ref[0, 0] = jnp.maximum(mx_ref[0, 0], bmx)


def _minmax(x2d):
    blk_rows = 512
    return pl.pallas_call(
        _minmax_body,
        grid=(ROWS // blk_rows,),
        in_specs=[pl.BlockSpec((blk_rows, COLS), lambda i: (i, 0))],
        out_specs=[
            pl.BlockSpec(memory_space=pltpu.SMEM),
            pl.BlockSpec(memory_space=pltpu.SMEM),
        ],
        out_shape=[
            jax.ShapeDtypeStruct((1, 1), jnp.float32),
            jax.ShapeDtypeStruct((1, 1), jnp.float32),
        ],
    )(x2d)


def _copy_body(x_ref, o_ref):
    o_ref[...] = x_ref[...]


def _copy(x2d):
    blk_rows = 512
    return pl.pallas_call(
        _copy_body,
        grid=(ROWS // blk_rows,),
        in_specs=[pl.BlockSpec((blk_rows, COLS), lambda i: (i, 0))],
        out_specs=pl.BlockSpec((blk_rows, COLS), lambda i: (i, 0)),
        out_shape=jax.ShapeDtypeStruct((ROWS, COLS), jnp.float32),
    )(x2d)


def _hist_sc_body(x_hbm, minv_hbm, invw_hbm, out_hbm,
                  buf0, buf1, buf2, buf3, prm0, prm1, hist, part,
                  sem0, sem1, sem2, sem3):
    bufs = (buf0, buf1, buf2, buf3)
    sems = (sem0, sem1, sem2, sem3)
    cid = lax.axis_index("c")
    sid = lax.axis_index("s")
    wid = sid * NC + cid
    row0 = wid * ROWS_PER_TILE

    pltpu.sync_copy(minv_hbm, prm0)
    pltpu.sync_copy(invw_hbm, prm1)
    minv = prm0[...]
    invw = prm1[...]
    nbias = jnp.zeros((L,), jnp.float32) - minv * invw
    lane = lax.iota(jnp.int32, L)
    lane16 = lane * L
    ones = jnp.full((L,), 1.0, jnp.float32)
    chi = jnp.full((L,), BINS - 1 + 0.5, jnp.float32)

    @plsc.parallel_loop(0, (L * BINS) // L, 1, unroll=4)
    def _zero(j):
        hist[pl.ds(j * L, L)] = jnp.zeros((L,), jnp.float32)

    for b in range(NBUF):
        pltpu.make_async_copy(
            x_hbm.at[pl.ds(row0 + b * CHUNK_ROWS, CHUNK_ROWS)],
            bufs[b], sems[b]).start()

    def _process(buf):
        @plsc.parallel_loop(0, VECS, 1, unroll=2)
        def _vec(i):
            for r in range(CHUNK_ROWS):
                v = buf[r, pl.ds(i * L, L)]
                q = v * invw + nbias
                iv = jnp.minimum(q, chi).astype(jnp.int32)
                addr = (iv << 4) + lane
                plsc.addupdate_scatter(hist, [addr], ones)

    def _round(g, c):
        for b in range(NBUF):
            ch = g * NBUF + b
            r_off = row0 + ch * CHUNK_ROWS
            pltpu.make_async_copy(
                x_hbm.at[pl.ds(r_off, CHUNK_ROWS)], bufs[b], sems[b]).wait()
            _process(bufs[b])

            @pl.when(ch + NBUF < NCHUNK)
            def _():
                pltpu.make_async_copy(
                    x_hbm.at[pl.ds(r_off + NBUF * CHUNK_ROWS, CHUNK_ROWS)],
                    bufs[b], sems[b]).start()

        return c

    lax.fori_loop(0, NCHUNK // NBUF, _round, 0)

    def _red(j, c):
        def _lane(l, acc):
            return acc + plsc.load_gather(hist, [lane16 + (j * (L * L) + l)])

        acc = lax.fori_loop(1, L, _lane,
                            plsc.load_gather(hist, [lane16 + j * (L * L)]))
        part[pl.ds(j * L, L)] = acc
        return c

    lax.fori_loop(0, BINS // L, _red, 0)

    pltpu.sync_copy(part, out_hbm.at[wid])


def _hist_sc(x2d, minv, invw):
    mesh = plsc.VectorSubcoreMesh(
        core_axis_name="c", subcore_axis_name="s",
        num_cores=NC, num_subcores=NS)
    call = functools.partial(
        pl.kernel,
        out_type=jax.ShapeDtypeStruct((NW, BINS), jnp.float32),
        mesh=mesh,
        compiler_params=pltpu.CompilerParams(needs_layout_passes=False),
        scratch_types=[
            pltpu.VMEM((CHUNK_ROWS, COLS), jnp.float32),
            pltpu.VMEM((CHUNK_ROWS, COLS), jnp.float32),
            pltpu.VMEM((CHUNK_ROWS, COLS), jnp.float32),
            pltpu.VMEM((CHUNK_ROWS, COLS), jnp.float32),
            pltpu.VMEM((L,), jnp.float32),
            pltpu.VMEM((L,), jnp.float32),
            pltpu.VMEM((L * BINS,), jnp.float32),
            pltpu.VMEM((BINS,), jnp.float32),
            pltpu.SemaphoreType.DMA,
            pltpu.SemaphoreType.DMA,
            pltpu.SemaphoreType.DMA,
            pltpu.SemaphoreType.DMA,
        ],
    )(_hist_sc_body)
    return call(x2d, minv, invw)


def kernel(x):
    x2d = x.reshape(ROWS, COLS)
    x_out = _copy(x2d)
    mn, mx = _minmax(x2d)
    min_val = mn[0, 0]
    max_val = mx[0, 0]
    width = (max_val - min_val) / jnp.float32(BINS)
    safe_width = jnp.where(width > 0, width, jnp.float32(1.0))
    invw = jnp.float32(1.0) / safe_width
    minv = jnp.full((L,), min_val, jnp.float32)
    invwv = jnp.full((L,), invw, jnp.float32)
    parts = _hist_sc(x2d, minv, invwv)
    hist = parts.sum(axis=0)
    return (x_out.reshape(x.shape), hist, min_val, max_val)
